# Initial kernel scaffold; baseline (speedup 1.0000x reference)
import jax
import jax.numpy as jnp
from jax.experimental import pallas as pl


def kernel(input, indices, update):
    raise NotImplementedError("write your pallas kernel here")


# speedup vs baseline: 4.2644x; 4.2644x over previous
import jax
import jax.numpy as jnp
from jax import lax
from jax.experimental import pallas as pl
from jax.experimental.pallas import tpu as pltpu
from jax.experimental.pallas import tpu_sc as plsc

D0, D1, D2, K = 100000, 8, 64, 16384
ROW = D1 * D2
NC, NS, L = 2, 16, 16
NW = NC * NS
OWN = (D0 + NW - 1) // NW
TSZ = ((OWN + L - 1) // L) * L
BS = 64
CAP = K + L
INF = jnp.int32(0x7FFFFFFF)


def _scatter_body(
    out_hbm, idx_hbm, upd_hbm, idxs, tbl, flat_k, flat_r, bk, br, rows, sem
):
    cid = lax.axis_index("c")
    sid = lax.axis_index("s")
    wid = sid * NC + cid
    lo = wid * OWN
    hi = jnp.minimum(lo + OWN, D0)
    lane = lax.iota(jnp.int32, L)

    pltpu.sync_copy(idx_hbm, idxs)

    @pl.loop(0, TSZ // L)
    def _clear(g):
        tbl[pl.ds(g * L, L)] = jnp.full((L,), -1, jnp.int32)

    @pl.loop(0, K // L)
    def _scan(g):
        vidx = idxs[pl.ds(g * L, L)]
        mask = (vidx >= lo) & (vidx < hi)
        kvec = g * L + lane
        addr = jnp.clip(vidx - lo, 0, OWN - 1)
        plsc.store_scatter(tbl, [addr], kvec, mask=mask)

        def _retry(active):
            got = plsc.load_gather(tbl, [addr], mask=mask)
            active = mask & (got < kvec)
            plsc.store_scatter(tbl, [addr], kvec, mask=active)
            return active

        lax.while_loop(jnp.any, _retry, mask)

    def _compress(g, cur):
        v = tbl[pl.ds(g * L, L)]
        m = v >= 0
        rowv = lo + g * L + lane
        plsc.store_compressed(flat_k.at[pl.ds(cur, L)], v, mask=m)
        plsc.store_compressed(flat_r.at[pl.ds(cur, L)], rowv, mask=m)
        return cur + plsc.all_reduce_population_count(m)[0]

    cnt = lax.fori_loop(0, TSZ // L, _compress, jnp.int32(0))

    @pl.when(cnt > 0)
    def _move():
        nb = (cnt + BS - 1) // BS

        klast = flat_k[pl.ds(cnt - 1, L)][0]
        rlast = flat_r[pl.ds(cnt - 1, L)][0]

        @pl.loop(cnt // L, (nb * BS) // L)
        def _pad(g):
            lanes = g * L + lane
            vk = flat_k[pl.ds(g * L, L)]
            vr = flat_r[pl.ds(g * L, L)]
            flat_k[pl.ds(g * L, L)] = jnp.where(lanes >= cnt, klast, vk)
            flat_r[pl.ds(g * L, L)] = jnp.where(lanes >= cnt, rlast, vr)

        @pl.loop(0, nb)
        def _batch(b):
            for i in range(BS // L):
                bk[pl.ds(i * L, L)] = flat_k[pl.ds(b * BS + i * L, L)]
                br[pl.ds(i * L, L)] = flat_r[pl.ds(b * BS + i * L, L)]
            pltpu.async_copy(upd_hbm.at[bk], rows, sem).wait()
            pltpu.async_copy(rows, out_hbm.at[br], sem).wait()


_scatter = pl.kernel(
    _scatter_body,
    out_type=(),
    mesh=plsc.VectorSubcoreMesh(core_axis_name="c", subcore_axis_name="s"),
    compiler_params=pltpu.CompilerParams(needs_layout_passes=False),
    scratch_types=[
        pltpu.VMEM((K,), jnp.int32),
        pltpu.VMEM((TSZ,), jnp.int32),
        pltpu.VMEM((CAP,), jnp.int32),
        pltpu.VMEM((CAP,), jnp.int32),
        pltpu.VMEM((BS,), jnp.int32),
        pltpu.VMEM((BS,), jnp.int32),
        pltpu.VMEM((BS, ROW), jnp.float32),
        pltpu.SemaphoreType.DMA,
    ],
)


def kernel(input, indices, update):
    inp2 = input.reshape(D0, ROW)
    upd2 = update.reshape(K, ROW)
    out_ref = jax.new_ref(inp2)
    _scatter(out_ref, indices, upd2)
    return out_ref[...].reshape(D0, D1, D2)


# speedup vs baseline: 4.2666x; 1.0005x over previous
import jax
import jax.numpy as jnp
from jax import lax
from jax.experimental import pallas as pl
from jax.experimental.pallas import tpu as pltpu
from jax.experimental.pallas import tpu_sc as plsc

D0, D1, D2, K = 100000, 8, 64, 16384
ROW = D1 * D2
NC, NS, L = 2, 16, 16
NW = NC * NS
OWN = (D0 + NW - 1) // NW
TSZ = ((OWN + L - 1) // L) * L
BS = 64
CAP = K + L


def _scatter_body(
    out_hbm, idx_hbm, upd_hbm, idxs, tbl, flat_k, flat_r, bk, br, rows, sem
):
    cid = lax.axis_index("c")
    sid = lax.axis_index("s")
    wid = sid * NC + cid
    lo = wid * OWN
    hi = jnp.minimum(lo + OWN, D0)
    lane = lax.iota(jnp.int32, L)

    pltpu.sync_copy(idx_hbm, idxs)

    @pl.loop(0, TSZ // L)
    def _clear(g):
        tbl[pl.ds(g * L, L)] = jnp.full((L,), -1, jnp.int32)

    @pl.loop(0, K // L)
    def _scan(g):
        vidx = idxs[pl.ds(g * L, L)]
        mask = (vidx >= lo) & (vidx < hi)
        kvec = g * L + lane
        addr = jnp.clip(vidx - lo, 0, OWN - 1)
        plsc.store_scatter(tbl, [addr], kvec, mask=mask)

        def _retry(active):
            got = plsc.load_gather(tbl, [addr], mask=mask)
            active = mask & (got < kvec)
            plsc.store_scatter(tbl, [addr], kvec, mask=active)
            return active

        lax.while_loop(jnp.any, _retry, mask)

    def _compress(g, cur):
        v = tbl[pl.ds(g * L, L)]
        m = v >= 0
        rowv = lo + g * L + lane
        plsc.store_compressed(flat_k.at[pl.ds(cur, L)], v, mask=m)
        plsc.store_compressed(flat_r.at[pl.ds(cur, L)], rowv, mask=m)
        return cur + plsc.all_reduce_population_count(m)[0]

    cnt = lax.fori_loop(0, TSZ // L, _compress, jnp.int32(0))

    @pl.when(cnt > 0)
    def _move():
        nb = (cnt + BS - 1) // BS

        klast = flat_k[pl.ds(cnt - 1, L)][0]
        rlast = flat_r[pl.ds(cnt - 1, L)][0]

        @pl.loop(cnt // L, (nb * BS) // L)
        def _pad(g):
            lanes = g * L + lane
            vk = flat_k[pl.ds(g * L, L)]
            vr = flat_r[pl.ds(g * L, L)]
            flat_k[pl.ds(g * L, L)] = jnp.where(lanes >= cnt, klast, vk)
            flat_r[pl.ds(g * L, L)] = jnp.where(lanes >= cnt, rlast, vr)

        @pl.loop(0, nb)
        def _batch(b):
            for i in range(BS // L):
                bk[pl.ds(i * L, L)] = flat_k[pl.ds(b * BS + i * L, L)]
                br[pl.ds(i * L, L)] = flat_r[pl.ds(b * BS + i * L, L)]
            pltpu.async_copy(upd_hbm.at[bk], rows, sem).wait()
            pltpu.async_copy(rows, out_hbm.at[br], sem).wait()


_scatter = pl.kernel(
    _scatter_body,
    out_type=(),
    mesh=plsc.VectorSubcoreMesh(core_axis_name="c", subcore_axis_name="s"),
    compiler_params=pltpu.CompilerParams(needs_layout_passes=False),
    scratch_types=[
        pltpu.VMEM((K,), jnp.int32),
        pltpu.VMEM((TSZ,), jnp.int32),
        pltpu.VMEM((CAP,), jnp.int32),
        pltpu.VMEM((CAP,), jnp.int32),
        pltpu.VMEM((BS,), jnp.int32),
        pltpu.VMEM((BS,), jnp.int32),
        pltpu.VMEM((BS, ROW), jnp.float32),
        pltpu.SemaphoreType.DMA,
    ],
)


def kernel(input, indices, update):
    inp2 = input.reshape(D0, ROW)
    upd2 = update.reshape(K, ROW)
    out_ref = jax.new_ref(inp2)
    _scatter(out_ref, indices, upd2)
    return jax.freeze(out_ref).reshape(D0, D1, D2)
